# baseline (speedup 1.0000x reference)
import jax
import jax.numpy as jnp
from jax import lax
from jax.experimental import pallas as pl
from jax.experimental.pallas import tpu as pltpu
from jax.experimental.pallas import tpu_sc as plsc

NUM_USERS = 25000
NUM_ITEMS = 25000
N_NODES = NUM_USERS + NUM_ITEMS
N_EDGES = 800000
DIM = 64
N_LAYERS = 3

HALF = N_NODES // 2
NUM_SUBCORES = 16
CHUNK = 128
SUPER = 14
NSUPER = 28
NCHUNKS = SUPER * NSUPER
BLK = SUPER * CHUNK
EPT = CHUNK * NCHUNKS
E_PAD = EPT * NUM_SUBCORES
NGROUPS = CHUNK // 16
ACC_ROWS = 25600
ZREG = ACC_ROWS // NUM_SUBCORES


def _layer_body(emb_hbm, src_hbm, dst_hbm, w_hbm, out_hbm,
                acc, rows0, rows1, loc0, loc1,
                bsrc0, bdst0, bval0, bsrc1, bdst1, bval1,
                sem_g0, sem_g1, sem_s0, sem_s1, sem_i0, sem_i1):
    c = lax.axis_index("c")
    s = lax.axis_index("s")
    base_node = c * HALF
    lane = lax.iota(jnp.int32, 16)
    zero16 = jnp.zeros((16,), jnp.float32)
    ebase = s * EPT

    rows = (rows0, rows1)
    locs = (loc0, loc1)
    sems_g = (sem_g0, sem_g1)
    sems_s = (sem_s0, sem_s1)
    sets = ((bsrc0, bdst0, bval0, sem_i0), (bsrc1, bdst1, bval1, sem_i1))

    def start_idx_load(b, st):
        bsrc, bdst, bval, sem = st
        off = ebase + b * BLK
        pltpu.async_copy(src_hbm.at[pl.ds(off, BLK)], bsrc, sem)
        pltpu.async_copy(dst_hbm.at[pl.ds(off, BLK)], bdst, sem)
        pltpu.async_copy(w_hbm.at[pl.ds(off, BLK)], bval, sem)

    def wait_idx_load(st):
        bsrc, bdst, bval, sem = st
        pltpu.make_async_copy(src_hbm.at[pl.ds(0, BLK)], bsrc, sem).wait()
        pltpu.make_async_copy(dst_hbm.at[pl.ds(0, BLK)], bdst, sem).wait()
        pltpu.make_async_copy(w_hbm.at[pl.ds(0, BLK)], bval, sem).wait()

    def start_gather(st, j, p):
        bsrc = st[0]
        pltpu.async_copy(emb_hbm.at[bsrc.at[pl.ds(j * CHUNK, CHUNK)]],
                         rows[p], sems_g[p])

    def wait_gather(p):
        pltpu.make_async_copy(emb_hbm.at[bsrc0.at[pl.ds(0, CHUNK)]],
                              rows[p], sems_g[p]).wait()

    def start_scatter(p):
        pltpu.async_copy(rows[p], acc.at[locs[p]], sems_s[p], add=True)

    def wait_scatter(p):
        pltpu.make_async_copy(rows[p], acc.at[locs[p]], sems_s[p]).wait()

    def scale(st, j, p):
        bdst, bval = st[1], st[2]
        ro, lo = rows[p], locs[p]

        @plsc.parallel_loop(0, NGROUPS, 1, unroll=4)
        def _(g):
            dvec = bdst[pl.ds(j * CHUNK + g * 16, 16)]
            dloc = dvec - base_node
            inr = (dloc >= 0) & (dloc < HALF)
            trash = HALF + g * 16 + lane
            lo[pl.ds(g * 16, 16)] = jnp.where(inr, dloc, trash)
            for jj in range(16):
                e = g * 16 + jj
                sp = plsc.load_gather(
                    bval, [jnp.full((16,), j * CHUNK + e, jnp.int32)])
                for d in range(4):
                    ro[e, pl.ds(d * 16, 16)] = ro[e, pl.ds(d * 16, 16)] * sp

    @pl.loop(0, CHUNK)
    def _(r):
        for d in range(4):
            rows0[r, pl.ds(d * 16, 16)] = zero16

    start_idx_load(0, sets[0])
    start_idx_load(1, sets[1])

    for k in range(12):
        pltpu.sync_copy(rows0, acc.at[pl.ds(s * ZREG + k * CHUNK, CHUNK)])
    pltpu.sync_copy(rows0.at[pl.ds(0, 64)],
                    acc.at[pl.ds(s * ZREG + 12 * CHUNK, 64)])
    plsc.subcore_barrier()

    wait_idx_load(sets[0])
    start_gather(sets[0], 0, 0)

    def make_chunk(cur_st, b, j, p, guard_first):
        wait_gather(p)
        scale(cur_st, j, p)
        if guard_first:
            @pl.when(b * SUPER + j > 0)
            def _():
                wait_scatter(1 - p)
        else:
            wait_scatter(1 - p)
        start_gather(cur_st, j + 1, 1 - p)
        start_scatter(p)

    def process_block(b, cur_st, nxt_st, has_next, load_next2):
        @pl.loop(0, (SUPER - 2) // 2)
        def _(hj):
            j = hj * 2
            make_chunk(cur_st, b, j, 0, True)
            make_chunk(cur_st, b, j + 1, 1, False)

        make_chunk(cur_st, b, SUPER - 2, 0, False)
        j = SUPER - 1
        p = 1
        wait_gather(p)
        scale(cur_st, j, p)
        wait_scatter(1 - p)

        def _handoff():
            wait_idx_load(nxt_st)
            start_gather(nxt_st, 0, 0)

        if has_next is True:
            _handoff()
        else:
            pl.when(has_next)(_handoff)

        @pl.when(load_next2)
        def _():
            start_idx_load(b + 2, cur_st)

        start_scatter(p)

    @pl.loop(0, NSUPER // 2)
    def _(bb):
        process_block(2 * bb, sets[0], sets[1], True, bb < NSUPER // 2 - 1)
        process_block(2 * bb + 1, sets[1], sets[0],
                      bb < NSUPER // 2 - 1, bb < NSUPER // 2 - 1)

    wait_scatter(1)
    plsc.subcore_barrier()

    @pl.when(s < 15)
    def _():
        fb = s * 1560
        pltpu.sync_copy(acc.at[pl.ds(fb, 1560)],
                        out_hbm.at[pl.ds(base_node + fb, 1560)])

    @pl.when(s == 15)
    def _():
        pltpu.sync_copy(acc.at[pl.ds(15 * 1560, 1600)],
                        out_hbm.at[pl.ds(base_node + 15 * 1560, 1600)])


_cp = pltpu.CompilerParams(needs_layout_passes=False, use_tc_tiling_on_sc=False)

_layer = pl.kernel(
    _layer_body,
    out_type=jax.ShapeDtypeStruct((N_NODES, DIM), jnp.float32),
    mesh=plsc.VectorSubcoreMesh(core_axis_name="c", subcore_axis_name="s"),
    compiler_params=_cp,
    scratch_types=[
        pltpu.VMEM_SHARED((ACC_ROWS, DIM), jnp.float32),
        pltpu.VMEM((CHUNK, DIM), jnp.float32),
        pltpu.VMEM((CHUNK, DIM), jnp.float32),
        pltpu.VMEM((CHUNK,), jnp.int32),
        pltpu.VMEM((CHUNK,), jnp.int32),
        pltpu.VMEM((BLK,), jnp.int32),
        pltpu.VMEM((BLK,), jnp.int32),
        pltpu.VMEM((BLK,), jnp.float32),
        pltpu.VMEM((BLK,), jnp.int32),
        pltpu.VMEM((BLK,), jnp.int32),
        pltpu.VMEM((BLK,), jnp.float32),
        pltpu.SemaphoreType.DMA,
        pltpu.SemaphoreType.DMA,
        pltpu.SemaphoreType.DMA,
        pltpu.SemaphoreType.DMA,
        pltpu.SemaphoreType.DMA,
        pltpu.SemaphoreType.DMA,
    ],
)


def kernel(user_emb_s, item_emb, edge_values, edge_index):
    all_emb = jnp.concatenate([user_emb_s, item_emb], axis=0)
    pad = E_PAD - N_EDGES
    src = jnp.concatenate([edge_index[0], jnp.zeros((pad,), jnp.int32)])
    dst = jnp.concatenate([edge_index[1], jnp.full((pad,), N_NODES, jnp.int32)])
    w = jnp.concatenate([edge_values, jnp.zeros((pad,), jnp.float32)])

    emb = all_emb
    total = all_emb
    for _ in range(N_LAYERS):
        emb = _layer(emb, src, dst, w)
        total = total + emb
    light_out = total * 0.25
    return light_out[:NUM_USERS], light_out[NUM_USERS:]


# speedup vs baseline: 1.2996x; 1.2996x over previous
import jax
import jax.numpy as jnp
from jax import lax
from jax.experimental import pallas as pl
from jax.experimental.pallas import tpu as pltpu
from jax.experimental.pallas import tpu_sc as plsc

NUM_USERS = 25000
NUM_ITEMS = 25000
N_NODES = NUM_USERS + NUM_ITEMS
N_EDGES = 800000
DIM = 64
HDIM = DIM // 2
N_LAYERS = 3

NUM_SUBCORES = 16
CHUNK = 128
SUPER = 14
NSUPER = 28
NCHUNKS = SUPER * NSUPER
BLK = SUPER * CHUNK
EPT = CHUNK * NCHUNKS
E_PAD = EPT * NUM_SUBCORES
NGROUPS = CHUNK // 16
ACC_ROWS = 50176
ZREG = ACC_ROWS // NUM_SUBCORES


def _layer_body(emb_hbm, src_hbm, dst_hbm, w_hbm, out_hbm,
                acc, rows0, rows1, loc0, loc1,
                bsrc0, bdst0, bval0, bsrc1, bdst1, bval1,
                sem_g0, sem_g1, sem_s0, sem_s1, sem_i0, sem_i1):
    c = lax.axis_index("c")
    s = lax.axis_index("s")
    zero16 = jnp.zeros((16,), jnp.float32)
    ebase = s * EPT
    emb_c = emb_hbm.at[c]
    out_c = out_hbm.at[c]

    rows = (rows0, rows1)
    locs = (loc0, loc1)
    sems_g = (sem_g0, sem_g1)
    sems_s = (sem_s0, sem_s1)
    sets = ((bsrc0, bdst0, bval0, sem_i0), (bsrc1, bdst1, bval1, sem_i1))

    def start_idx_load(b, st):
        bsrc, bdst, bval, sem = st
        off = ebase + b * BLK
        pltpu.async_copy(src_hbm.at[pl.ds(off, BLK)], bsrc, sem)
        pltpu.async_copy(dst_hbm.at[pl.ds(off, BLK)], bdst, sem)
        pltpu.async_copy(w_hbm.at[pl.ds(off, BLK)], bval, sem)

    def wait_idx_load(st):
        bsrc, bdst, bval, sem = st
        pltpu.make_async_copy(src_hbm.at[pl.ds(0, BLK)], bsrc, sem).wait()
        pltpu.make_async_copy(dst_hbm.at[pl.ds(0, BLK)], bdst, sem).wait()
        pltpu.make_async_copy(w_hbm.at[pl.ds(0, BLK)], bval, sem).wait()

    def start_gather(st, j, p):
        bsrc = st[0]
        pltpu.async_copy(emb_c.at[bsrc.at[pl.ds(j * CHUNK, CHUNK)]],
                         rows[p], sems_g[p])

    def wait_gather(p):
        pltpu.make_async_copy(emb_c.at[bsrc0.at[pl.ds(0, CHUNK)]],
                              rows[p], sems_g[p]).wait()

    def start_scatter(p):
        pltpu.async_copy(rows[p], acc.at[locs[p]], sems_s[p], add=True)

    def wait_scatter(p):
        pltpu.make_async_copy(rows[p], acc.at[locs[p]], sems_s[p]).wait()

    def scale(st, j, p):
        bdst, bval = st[1], st[2]
        ro, lo = rows[p], locs[p]

        @plsc.parallel_loop(0, NGROUPS, 1, unroll=2)
        def _(g):
            lo[pl.ds(g * 16, 16)] = bdst[pl.ds(j * CHUNK + g * 16, 16)]
            for jj in range(16):
                e = g * 16 + jj
                sp = plsc.load_gather(
                    bval, [jnp.full((16,), j * CHUNK + e, jnp.int32)])
                for d in range(2):
                    ro[e, pl.ds(d * 16, 16)] = ro[e, pl.ds(d * 16, 16)] * sp

    @pl.loop(0, CHUNK)
    def _(r):
        for d in range(2):
            rows0[r, pl.ds(d * 16, 16)] = zero16

    start_idx_load(0, sets[0])
    start_idx_load(1, sets[1])

    for k in range(24):
        pltpu.sync_copy(rows0, acc.at[pl.ds(s * ZREG + k * CHUNK, CHUNK)])
    pltpu.sync_copy(rows0.at[pl.ds(0, 64)],
                    acc.at[pl.ds(s * ZREG + 24 * CHUNK, 64)])
    plsc.subcore_barrier()

    wait_idx_load(sets[0])
    start_gather(sets[0], 0, 0)

    def make_chunk(cur_st, b, j, p, guard_first):
        wait_gather(p)
        scale(cur_st, j, p)
        if guard_first:
            @pl.when(b * SUPER + j > 0)
            def _():
                wait_scatter(1 - p)
        else:
            wait_scatter(1 - p)
        start_gather(cur_st, j + 1, 1 - p)
        start_scatter(p)

    def process_block(b, cur_st, nxt_st, has_next, load_next2):
        @pl.loop(0, (SUPER - 2) // 2)
        def _(hj):
            j = hj * 2
            make_chunk(cur_st, b, j, 0, True)
            make_chunk(cur_st, b, j + 1, 1, False)

        make_chunk(cur_st, b, SUPER - 2, 0, False)
        j = SUPER - 1
        p = 1
        wait_gather(p)
        scale(cur_st, j, p)
        wait_scatter(1 - p)

        def _handoff():
            wait_idx_load(nxt_st)
            start_gather(nxt_st, 0, 0)

        if has_next is True:
            _handoff()
        else:
            pl.when(has_next)(_handoff)

        @pl.when(load_next2)
        def _():
            start_idx_load(b + 2, cur_st)

        start_scatter(p)

    @pl.loop(0, NSUPER // 2)
    def _(bb):
        process_block(2 * bb, sets[0], sets[1], True, bb < NSUPER // 2 - 1)
        process_block(2 * bb + 1, sets[1], sets[0],
                      bb < NSUPER // 2 - 1, bb < NSUPER // 2 - 1)

    wait_scatter(1)
    plsc.subcore_barrier()

    @pl.when(s < 15)
    def _():
        fb = s * 3120
        pltpu.sync_copy(acc.at[pl.ds(fb, 3120)], out_c.at[pl.ds(fb, 3120)])

    @pl.when(s == 15)
    def _():
        pltpu.sync_copy(acc.at[pl.ds(15 * 3120, 3200)],
                        out_c.at[pl.ds(15 * 3120, 3200)])


_cp = pltpu.CompilerParams(needs_layout_passes=False, use_tc_tiling_on_sc=False)

_layer = pl.kernel(
    _layer_body,
    out_type=jax.ShapeDtypeStruct((2, N_NODES, HDIM), jnp.float32),
    mesh=plsc.VectorSubcoreMesh(core_axis_name="c", subcore_axis_name="s"),
    compiler_params=_cp,
    scratch_types=[
        pltpu.VMEM_SHARED((ACC_ROWS, HDIM), jnp.float32),
        pltpu.VMEM((CHUNK, HDIM), jnp.float32),
        pltpu.VMEM((CHUNK, HDIM), jnp.float32),
        pltpu.VMEM((CHUNK,), jnp.int32),
        pltpu.VMEM((CHUNK,), jnp.int32),
        pltpu.VMEM((BLK,), jnp.int32),
        pltpu.VMEM((BLK,), jnp.int32),
        pltpu.VMEM((BLK,), jnp.float32),
        pltpu.VMEM((BLK,), jnp.int32),
        pltpu.VMEM((BLK,), jnp.int32),
        pltpu.VMEM((BLK,), jnp.float32),
        pltpu.SemaphoreType.DMA,
        pltpu.SemaphoreType.DMA,
        pltpu.SemaphoreType.DMA,
        pltpu.SemaphoreType.DMA,
        pltpu.SemaphoreType.DMA,
        pltpu.SemaphoreType.DMA,
    ],
)


def kernel(user_emb_s, item_emb, edge_values, edge_index):
    all_emb = jnp.concatenate([user_emb_s, item_emb], axis=0)
    emb2 = jnp.stack([all_emb[:, :HDIM], all_emb[:, HDIM:]])
    pad = E_PAD - N_EDGES
    src = jnp.concatenate([edge_index[0], jnp.zeros((pad,), jnp.int32)])
    dst = jnp.concatenate(
        [edge_index[1],
         N_NODES + (jnp.arange(pad, dtype=jnp.int32) % CHUNK)])
    w = jnp.concatenate([edge_values, jnp.zeros((pad,), jnp.float32)])

    emb = emb2
    total = emb2
    for _ in range(N_LAYERS):
        emb = _layer(emb, src, dst, w)
        total = total + emb
    light_out = jnp.concatenate([total[0], total[1]], axis=1) * 0.25
    return light_out[:NUM_USERS], light_out[NUM_USERS:]
